# baseline (speedup 1.0000x reference)
import functools

import jax
import jax.numpy as jnp
import numpy as np
from jax.experimental import pallas as pl
from jax.experimental.pallas import tpu as pltpu

T = 100
DEPTH = 10
NPT = 2 ** DEPTH - 1
H = 128
IN = 128
B = 4


def _tree_gru_kernel(x_ref, wih_ref, whh_ref, bih_ref, bhh_ref, out_ref):
    wih = wih_ref[...]
    whh = whh_ref[...]
    bih = bih_ref[...]
    bhh = bhh_ref[...]

    x0 = x_ref[:, 0, :]
    gx = jnp.dot(x0, wih, preferred_element_type=jnp.float32) + bih
    r = jax.nn.sigmoid(gx[:, :H] + bhh[:, :H])
    z = jax.nn.sigmoid(gx[:, H:2 * H] + bhh[:, H:2 * H])
    n = jnp.tanh(gx[:, 2 * H:] + r * bhh[:, 2 * H:])
    h = ((1.0 - z) * n).reshape(B, 1, H)

    for l in range(1, DEPTH):
        npar = 2 ** (l - 1)
        nl = 2 ** l
        hp = h.reshape(B * npar, H)
        gh = jnp.dot(hp, whh, preferred_element_type=jnp.float32) + bhh
        gh2 = jnp.repeat(gh.reshape(B, npar, 3 * H), 2, axis=1)
        gh2 = gh2.reshape(B * nl, 3 * H)
        hpar = jnp.repeat(h, 2, axis=1).reshape(B * nl, H)
        x = x_ref[:, nl - 1:2 * nl - 1, :].reshape(B * nl, IN)
        gx = jnp.dot(x, wih, preferred_element_type=jnp.float32) + bih
        r = jax.nn.sigmoid(gx[:, :H] + gh2[:, :H])
        z = jax.nn.sigmoid(gx[:, H:2 * H] + gh2[:, H:2 * H])
        n = jnp.tanh(gx[:, 2 * H:] + r * gh2[:, 2 * H:])
        h = ((1.0 - z) * n + z * hpar).reshape(B, nl, H)

    out_ref[0] = jnp.max(h, axis=1)


@functools.partial(jax.jit, static_argnames=())
def kernel(inputs, W_ih, W_hh, b_ih, b_hh, parent):
    del parent
    x = inputs.reshape(T, NPT, IN)
    wih_t = W_ih.T
    whh_t = W_hh.T
    bih = b_ih.reshape(1, 3 * H)
    bhh = b_hh.reshape(1, 3 * H)

    grid = (T // B,)
    return pl.pallas_call(
        _tree_gru_kernel,
        grid=grid,
        in_specs=[
            pl.BlockSpec((B, NPT, IN), lambda i: (i, 0, 0)),
            pl.BlockSpec((IN, 3 * H), lambda i: (0, 0)),
            pl.BlockSpec((H, 3 * H), lambda i: (0, 0)),
            pl.BlockSpec((1, 3 * H), lambda i: (0, 0)),
            pl.BlockSpec((1, 3 * H), lambda i: (0, 0)),
        ],
        out_specs=pl.BlockSpec((1, B, H), lambda i: (i, 0, 0)),
        out_shape=jax.ShapeDtypeStruct((T // B, B, H), jnp.float32),
        compiler_params=pltpu.CompilerParams(
            dimension_semantics=("parallel",),
        ),
    )(x, wih_t, whh_t, bih, bhh).reshape(T, H)


# speedup vs baseline: 1.8350x; 1.8350x over previous
import functools

import jax
import jax.numpy as jnp
import numpy as np
from jax.experimental import pallas as pl
from jax.experimental.pallas import tpu as pltpu

T = 100
DEPTH = 10
NPT = 2 ** DEPTH - 1
H = 128
IN = 128
B = 10


def _tree_gru_kernel(x_ref, wih_ref, whh_ref, bih_ref, bhh_ref, out_ref):
    wih = wih_ref[...]
    whh = whh_ref[...]
    bih = bih_ref[...]
    bhh = bhh_ref[...]

    x0 = x_ref[:, 0, :]
    gx = jnp.dot(x0, wih, preferred_element_type=jnp.float32) + bih
    r = jax.nn.sigmoid(gx[:, :H] + bhh[:, :H])
    z = jax.nn.sigmoid(gx[:, H:2 * H] + bhh[:, H:2 * H])
    n = jnp.tanh(gx[:, 2 * H:] + r * bhh[:, 2 * H:])
    h = ((1.0 - z) * n).reshape(B, 1, H)

    for l in range(1, DEPTH):
        nl = 2 ** l
        hpar = jnp.repeat(h, 2, axis=1).reshape(B * nl, H)
        gh = jnp.dot(hpar, whh, preferred_element_type=jnp.float32) + bhh
        x = x_ref[:, nl - 1:2 * nl - 1, :].reshape(B * nl, IN)
        gx = jnp.dot(x, wih, preferred_element_type=jnp.float32) + bih
        r = jax.nn.sigmoid(gx[:, :H] + gh[:, :H])
        z = jax.nn.sigmoid(gx[:, H:2 * H] + gh[:, H:2 * H])
        n = jnp.tanh(gx[:, 2 * H:] + r * gh[:, 2 * H:])
        h = ((1.0 - z) * n + z * hpar).reshape(B, nl, H)

    out_ref[0] = jnp.max(h, axis=1)


@functools.partial(jax.jit, static_argnames=())
def kernel(inputs, W_ih, W_hh, b_ih, b_hh, parent):
    del parent
    x = inputs.reshape(T, NPT, IN)
    wih_t = W_ih.T
    whh_t = W_hh.T
    bih = b_ih.reshape(1, 3 * H)
    bhh = b_hh.reshape(1, 3 * H)

    grid = (T // B,)
    return pl.pallas_call(
        _tree_gru_kernel,
        grid=grid,
        in_specs=[
            pl.BlockSpec((B, NPT, IN), lambda i: (i, 0, 0)),
            pl.BlockSpec((IN, 3 * H), lambda i: (0, 0)),
            pl.BlockSpec((H, 3 * H), lambda i: (0, 0)),
            pl.BlockSpec((1, 3 * H), lambda i: (0, 0)),
            pl.BlockSpec((1, 3 * H), lambda i: (0, 0)),
        ],
        out_specs=pl.BlockSpec((1, B, H), lambda i: (i, 0, 0)),
        out_shape=jax.ShapeDtypeStruct((T // B, B, H), jnp.float32),
        compiler_params=pltpu.CompilerParams(
            dimension_semantics=("parallel",),
        ),
    )(x, wih_t, whh_t, bih, bhh).reshape(T, H)


# speedup vs baseline: 2.2217x; 1.2107x over previous
import functools

import jax
import jax.numpy as jnp
from jax.experimental import pallas as pl
from jax.experimental.pallas import tpu as pltpu

T = 100
DEPTH = 10
NPT = 2 ** DEPTH - 1
H = 128
IN = 128
B = 10


def _sigmoid(x):
    return 0.5 + 0.5 * jnp.tanh(0.5 * x)


def _tree_gru_kernel(x_ref, wih_ref, whh_ref, bih_ref, bhh_ref, out_ref,
                     h_scr):
    wih = wih_ref[...]
    whh = whh_ref[...]
    bih = bih_ref[...]
    bhh = bhh_ref[...]

    x0 = x_ref[:, 0, :]
    gx = jnp.dot(x0, wih, preferred_element_type=jnp.float32) + bih
    r = _sigmoid(gx[:, :H] + bhh[:, :H])
    z = _sigmoid(gx[:, H:2 * H] + bhh[:, H:2 * H])
    n = jnp.tanh(gx[:, 2 * H:] + r * bhh[:, 2 * H:])
    h_scr[:, 0:1, :] = ((1.0 - z) * n).reshape(B, 1, H)

    pooled = None
    for l in range(1, DEPTH):
        m = 2 ** (l - 1)
        nl = 2 ** l
        hp = h_scr[:, m - 1:2 * m - 1, :].reshape(B * m, H)
        gh = jnp.dot(hp, whh, preferred_element_type=jnp.float32) + bhh
        halves = []
        for s in (0, 1):
            x = x_ref[:, nl - 1 + s:2 * nl - 1:2, :].reshape(B * m, IN)
            gx = jnp.dot(x, wih, preferred_element_type=jnp.float32) + bih
            r = _sigmoid(gx[:, :H] + gh[:, :H])
            z = _sigmoid(gx[:, H:2 * H] + gh[:, H:2 * H])
            n = jnp.tanh(gx[:, 2 * H:] + r * gh[:, 2 * H:])
            halves.append(n + z * (hp - n))
        if l < DEPTH - 1:
            h_scr[:, nl - 1:2 * nl - 1:2, :] = halves[0].reshape(B, m, H)
            h_scr[:, nl:2 * nl - 1:2, :] = halves[1].reshape(B, m, H)
        else:
            mL = jnp.max(halves[0].reshape(B, m, H), axis=1)
            mR = jnp.max(halves[1].reshape(B, m, H), axis=1)
            pooled = jnp.maximum(mL, mR)

    out_ref[0] = pooled


@functools.partial(jax.jit, static_argnames=())
def kernel(inputs, W_ih, W_hh, b_ih, b_hh, parent):
    del parent
    x = inputs.reshape(T, NPT, IN)
    wih_t = W_ih.T
    whh_t = W_hh.T
    bih = b_ih.reshape(1, 3 * H)
    bhh = b_hh.reshape(1, 3 * H)

    grid = (T // B,)
    return pl.pallas_call(
        _tree_gru_kernel,
        grid=grid,
        in_specs=[
            pl.BlockSpec((B, NPT, IN), lambda i: (i, 0, 0)),
            pl.BlockSpec((IN, 3 * H), lambda i: (0, 0)),
            pl.BlockSpec((H, 3 * H), lambda i: (0, 0)),
            pl.BlockSpec((1, 3 * H), lambda i: (0, 0)),
            pl.BlockSpec((1, 3 * H), lambda i: (0, 0)),
        ],
        out_specs=pl.BlockSpec((1, B, H), lambda i: (i, 0, 0)),
        out_shape=jax.ShapeDtypeStruct((T // B, B, H), jnp.float32),
        scratch_shapes=[pltpu.VMEM((B, NPT, H), jnp.float32)],
        compiler_params=pltpu.CompilerParams(
            dimension_semantics=("parallel",),
        ),
    )(x, wih_t, whh_t, bih, bhh).reshape(T, H)


# speedup vs baseline: 2.4534x; 1.1043x over previous
import functools

import jax
import jax.numpy as jnp
from jax.experimental import pallas as pl
from jax.experimental.pallas import tpu as pltpu

T = 100
DEPTH = 10
NPT = 2 ** DEPTH - 1
H = 128
IN = 128
B = 10


def _sigmoid(x):
    return 0.5 + 0.5 * jnp.tanh(0.5 * x)


def _tree_gru_kernel(x_hbm, wih_ref, whh_ref, bih_ref, bhh_ref, out_ref,
                     x_scr, h_scr, sem):
    i = pl.program_id(0)
    copies = [
        pltpu.make_async_copy(
            x_hbm.at[pl.ds((i * B + t) * NPT, NPT), :], x_scr.at[t], sem)
        for t in range(B)
    ]
    for c in copies:
        c.start()
    for c in copies:
        c.wait()

    wih = wih_ref[...]
    whh = whh_ref[...]
    bih = bih_ref[...]
    bhh = bhh_ref[...]

    x0 = x_scr[:, 0, :]
    gx = jnp.dot(x0, wih, preferred_element_type=jnp.float32) + bih
    r = _sigmoid(gx[:, :H] + bhh[:, :H])
    z = _sigmoid(gx[:, H:2 * H] + bhh[:, H:2 * H])
    n = jnp.tanh(gx[:, 2 * H:] + r * bhh[:, 2 * H:])
    h_scr[:, 0:1, :] = ((1.0 - z) * n).reshape(B, 1, H)

    pooled = None
    for l in range(1, DEPTH):
        m = 2 ** (l - 1)
        nl = 2 ** l
        hp = h_scr[:, m - 1:2 * m - 1, :].reshape(B * m, H)
        gh = jnp.dot(hp, whh, preferred_element_type=jnp.float32) + bhh
        halves = []
        for s in (0, 1):
            x = x_scr[:, nl - 1 + s:2 * nl - 1:2, :].reshape(B * m, IN)
            gx = jnp.dot(x, wih, preferred_element_type=jnp.float32) + bih
            r = _sigmoid(gx[:, :H] + gh[:, :H])
            z = _sigmoid(gx[:, H:2 * H] + gh[:, H:2 * H])
            n = jnp.tanh(gx[:, 2 * H:] + r * gh[:, 2 * H:])
            halves.append(n + z * (hp - n))
        if l < DEPTH - 1:
            h_scr[:, nl - 1:2 * nl - 1:2, :] = halves[0].reshape(B, m, H)
            h_scr[:, nl:2 * nl - 1:2, :] = halves[1].reshape(B, m, H)
        else:
            mL = jnp.max(halves[0].reshape(B, m, H), axis=1)
            mR = jnp.max(halves[1].reshape(B, m, H), axis=1)
            pooled = jnp.maximum(mL, mR)

    out_ref[0] = pooled


@functools.partial(jax.jit, static_argnames=())
def kernel(inputs, W_ih, W_hh, b_ih, b_hh, parent):
    del parent
    wih_t = W_ih.T
    whh_t = W_hh.T
    bih = b_ih.reshape(1, 3 * H)
    bhh = b_hh.reshape(1, 3 * H)

    grid = (T // B,)
    return pl.pallas_call(
        _tree_gru_kernel,
        grid=grid,
        in_specs=[
            pl.BlockSpec(memory_space=pl.ANY),
            pl.BlockSpec((IN, 3 * H), lambda i: (0, 0)),
            pl.BlockSpec((H, 3 * H), lambda i: (0, 0)),
            pl.BlockSpec((1, 3 * H), lambda i: (0, 0)),
            pl.BlockSpec((1, 3 * H), lambda i: (0, 0)),
        ],
        out_specs=pl.BlockSpec((1, B, H), lambda i: (i, 0, 0)),
        out_shape=jax.ShapeDtypeStruct((T // B, B, H), jnp.float32),
        scratch_shapes=[
            pltpu.VMEM((B, NPT, IN), jnp.float32),
            pltpu.VMEM((B, NPT, H), jnp.float32),
            pltpu.SemaphoreType.DMA,
        ],
        compiler_params=pltpu.CompilerParams(
            dimension_semantics=("arbitrary",),
        ),
    )(inputs, wih_t, whh_t, bih, bhh).reshape(T, H)


# speedup vs baseline: 3.2306x; 1.3168x over previous
import functools

import jax
import jax.numpy as jnp
from jax.experimental import pallas as pl
from jax.experimental.pallas import tpu as pltpu

T = 100
DEPTH = 10
NPT = 2 ** DEPTH - 1
H = 128
IN = 128
B = 10


def _sigmoid(x):
    return 0.5 + 0.5 * jnp.tanh(0.5 * x)


def _tree_gru_kernel(x_hbm, wih_ref, whh_ref, bih_ref, bhh_ref, out_ref,
                     x_scr, h_scr, sem):
    i = pl.program_id(0)

    def start_copies(prog, slot):
        for t in range(B):
            pltpu.make_async_copy(
                x_hbm.at[pl.ds((prog * B + t) * NPT, NPT), :],
                x_scr.at[slot, t], sem.at[slot]).start()

    @pl.when(i == 0)
    def _():
        start_copies(0, 0)

    @pl.when(i + 1 < pl.num_programs(0))
    def _():
        start_copies(i + 1, (i + 1) % 2)

    slot = i % 2
    for t in range(B):
        pltpu.make_async_copy(
            x_hbm.at[pl.ds(t * NPT, NPT), :],
            x_scr.at[slot, t], sem.at[slot]).wait()
    x_cur = x_scr.at[slot]

    wih = wih_ref[...]
    whh = whh_ref[...]
    bih = bih_ref[...]
    bhh = bhh_ref[...]

    x0 = x_cur[:, 0, :]
    gx = jnp.dot(x0, wih, preferred_element_type=jnp.float32) + bih
    r = _sigmoid(gx[:, :H] + bhh[:, :H])
    z = _sigmoid(gx[:, H:2 * H] + bhh[:, H:2 * H])
    n = jnp.tanh(gx[:, 2 * H:] + r * bhh[:, 2 * H:])
    h_scr[:, 0:1, :] = ((1.0 - z) * n).reshape(B, 1, H)

    pooled = None
    for l in range(1, DEPTH):
        m = 2 ** (l - 1)
        nl = 2 ** l
        hp = h_scr[:, m - 1:2 * m - 1, :].reshape(B * m, H)
        gh = jnp.dot(hp, whh, preferred_element_type=jnp.float32) + bhh
        halves = []
        for s in (0, 1):
            x = x_cur[:, nl - 1 + s:2 * nl - 1:2, :].reshape(B * m, IN)
            gx = jnp.dot(x, wih, preferred_element_type=jnp.float32) + bih
            r = _sigmoid(gx[:, :H] + gh[:, :H])
            z = _sigmoid(gx[:, H:2 * H] + gh[:, H:2 * H])
            n = jnp.tanh(gx[:, 2 * H:] + r * gh[:, 2 * H:])
            halves.append(n + z * (hp - n))
        if l < DEPTH - 1:
            h_scr[:, nl - 1:2 * nl - 1:2, :] = halves[0].reshape(B, m, H)
            h_scr[:, nl:2 * nl - 1:2, :] = halves[1].reshape(B, m, H)
        else:
            mL = jnp.max(halves[0].reshape(B, m, H), axis=1)
            mR = jnp.max(halves[1].reshape(B, m, H), axis=1)
            pooled = jnp.maximum(mL, mR)

    out_ref[0] = pooled


@functools.partial(jax.jit, static_argnames=())
def kernel(inputs, W_ih, W_hh, b_ih, b_hh, parent):
    del parent
    wih_t = W_ih.T
    whh_t = W_hh.T
    bih = b_ih.reshape(1, 3 * H)
    bhh = b_hh.reshape(1, 3 * H)

    grid = (T // B,)
    return pl.pallas_call(
        _tree_gru_kernel,
        grid=grid,
        in_specs=[
            pl.BlockSpec(memory_space=pl.ANY),
            pl.BlockSpec((IN, 3 * H), lambda i: (0, 0)),
            pl.BlockSpec((H, 3 * H), lambda i: (0, 0)),
            pl.BlockSpec((1, 3 * H), lambda i: (0, 0)),
            pl.BlockSpec((1, 3 * H), lambda i: (0, 0)),
        ],
        out_specs=pl.BlockSpec((1, B, H), lambda i: (i, 0, 0)),
        out_shape=jax.ShapeDtypeStruct((T // B, B, H), jnp.float32),
        scratch_shapes=[
            pltpu.VMEM((2, B, NPT, IN), jnp.float32),
            pltpu.VMEM((B, NPT, H), jnp.float32),
            pltpu.SemaphoreType.DMA((2,)),
        ],
        compiler_params=pltpu.CompilerParams(
            dimension_semantics=("arbitrary",),
        ),
    )(inputs, wih_t, whh_t, bih, bhh).reshape(T, H)


# speedup vs baseline: 3.5243x; 1.0909x over previous
import functools

import jax
import jax.numpy as jnp
from jax.experimental import pallas as pl
from jax.experimental.pallas import tpu as pltpu

T = 100
DEPTH = 10
NPT = 2 ** DEPTH - 1
H = 128
IN = 128
B = 20


def _sigmoid(x):
    return 0.5 + 0.5 * jnp.tanh(0.5 * x)


def _tree_gru_kernel(x_hbm, wih_ref, whh_ref, bih_ref, bhh_ref, out_ref,
                     x_scr, h_scr, sem):
    i = pl.program_id(0)

    def start_copies(prog, slot):
        for t in range(B):
            pltpu.make_async_copy(
                x_hbm.at[pl.ds((prog * B + t) * NPT, NPT), :],
                x_scr.at[slot, t], sem.at[slot]).start()

    @pl.when(i == 0)
    def _():
        start_copies(0, 0)

    @pl.when(i + 1 < pl.num_programs(0))
    def _():
        start_copies(i + 1, (i + 1) % 2)

    slot = i % 2
    for t in range(B):
        pltpu.make_async_copy(
            x_hbm.at[pl.ds(t * NPT, NPT), :],
            x_scr.at[slot, t], sem.at[slot]).wait()
    x_cur = x_scr.at[slot]

    wih = wih_ref[...]
    whh = whh_ref[...]
    bih = bih_ref[...]
    bhh = bhh_ref[...]

    x0 = x_cur[:, 0, :]
    gx = jnp.dot(x0, wih, preferred_element_type=jnp.float32) + bih
    r = _sigmoid(gx[:, :H] + bhh[:, :H])
    z = _sigmoid(gx[:, H:2 * H] + bhh[:, H:2 * H])
    n = jnp.tanh(gx[:, 2 * H:] + r * bhh[:, 2 * H:])
    h_scr[:, 0:1, :] = ((1.0 - z) * n).reshape(B, 1, H)

    pooled = None
    for l in range(1, DEPTH):
        m = 2 ** (l - 1)
        nl = 2 ** l
        hp = h_scr[:, m - 1:2 * m - 1, :].reshape(B * m, H)
        gh = jnp.dot(hp, whh, preferred_element_type=jnp.float32) + bhh
        halves = []
        for s in (0, 1):
            x = x_cur[:, nl - 1 + s:2 * nl - 1:2, :].reshape(B * m, IN)
            gx = jnp.dot(x, wih, preferred_element_type=jnp.float32) + bih
            r = _sigmoid(gx[:, :H] + gh[:, :H])
            z = _sigmoid(gx[:, H:2 * H] + gh[:, H:2 * H])
            n = jnp.tanh(gx[:, 2 * H:] + r * gh[:, 2 * H:])
            halves.append(n + z * (hp - n))
        if l < DEPTH - 1:
            h_scr[:, nl - 1:2 * nl - 1:2, :] = halves[0].reshape(B, m, H)
            h_scr[:, nl:2 * nl - 1:2, :] = halves[1].reshape(B, m, H)
        else:
            mL = jnp.max(halves[0].reshape(B, m, H), axis=1)
            mR = jnp.max(halves[1].reshape(B, m, H), axis=1)
            pooled = jnp.maximum(mL, mR)

    out_ref[0] = pooled


@functools.partial(jax.jit, static_argnames=())
def kernel(inputs, W_ih, W_hh, b_ih, b_hh, parent):
    del parent
    wih_t = W_ih.T
    whh_t = W_hh.T
    bih = b_ih.reshape(1, 3 * H)
    bhh = b_hh.reshape(1, 3 * H)

    grid = (T // B,)
    return pl.pallas_call(
        _tree_gru_kernel,
        grid=grid,
        in_specs=[
            pl.BlockSpec(memory_space=pl.ANY),
            pl.BlockSpec((IN, 3 * H), lambda i: (0, 0)),
            pl.BlockSpec((H, 3 * H), lambda i: (0, 0)),
            pl.BlockSpec((1, 3 * H), lambda i: (0, 0)),
            pl.BlockSpec((1, 3 * H), lambda i: (0, 0)),
        ],
        out_specs=pl.BlockSpec((1, B, H), lambda i: (i, 0, 0)),
        out_shape=jax.ShapeDtypeStruct((T // B, B, H), jnp.float32),
        scratch_shapes=[
            pltpu.VMEM((2, B, NPT, IN), jnp.float32),
            pltpu.VMEM((B, NPT, H), jnp.float32),
            pltpu.SemaphoreType.DMA((2,)),
        ],
        compiler_params=pltpu.CompilerParams(
            dimension_semantics=("arbitrary",),
        ),
    )(inputs, wih_t, whh_t, bih, bhh).reshape(T, H)


# speedup vs baseline: 3.5383x; 1.0040x over previous
import functools

import jax
import jax.numpy as jnp
from jax.experimental import pallas as pl
from jax.experimental.pallas import tpu as pltpu

T = 100
DEPTH = 10
NPT = 2 ** DEPTH - 1
H = 128
IN = 128
B = 25


def _sigmoid(x):
    return 0.5 + 0.5 * jnp.tanh(0.5 * x)


def _tree_gru_kernel(x_hbm, wih_ref, whh_ref, bih_ref, bhh_ref, out_ref,
                     x_scr, h_scr, sem):
    i = pl.program_id(0)

    def start_copies(prog, slot):
        for t in range(B):
            pltpu.make_async_copy(
                x_hbm.at[pl.ds((prog * B + t) * NPT, NPT), :],
                x_scr.at[slot, t], sem.at[slot]).start()

    @pl.when(i == 0)
    def _():
        start_copies(0, 0)

    @pl.when(i + 1 < pl.num_programs(0))
    def _():
        start_copies(i + 1, (i + 1) % 2)

    slot = i % 2
    for t in range(B):
        pltpu.make_async_copy(
            x_hbm.at[pl.ds(t * NPT, NPT), :],
            x_scr.at[slot, t], sem.at[slot]).wait()
    x_cur = x_scr.at[slot]

    wih = wih_ref[...]
    whh = whh_ref[...]
    bih = bih_ref[...]
    bhh = bhh_ref[...]

    x0 = x_cur[:, 0, :]
    gx = jnp.dot(x0, wih, preferred_element_type=jnp.float32) + bih
    r = _sigmoid(gx[:, :H] + bhh[:, :H])
    z = _sigmoid(gx[:, H:2 * H] + bhh[:, H:2 * H])
    n = jnp.tanh(gx[:, 2 * H:] + r * bhh[:, 2 * H:])
    h_scr[:, 0:1, :] = ((1.0 - z) * n).reshape(B, 1, H)

    pooled = None
    for l in range(1, DEPTH):
        m = 2 ** (l - 1)
        nl = 2 ** l
        hp = h_scr[:, m - 1:2 * m - 1, :].reshape(B * m, H)
        gh = jnp.dot(hp, whh, preferred_element_type=jnp.float32) + bhh
        halves = []
        for s in (0, 1):
            x = x_cur[:, nl - 1 + s:2 * nl - 1:2, :].reshape(B * m, IN)
            gx = jnp.dot(x, wih, preferred_element_type=jnp.float32) + bih
            r = _sigmoid(gx[:, :H] + gh[:, :H])
            z = _sigmoid(gx[:, H:2 * H] + gh[:, H:2 * H])
            n = jnp.tanh(gx[:, 2 * H:] + r * gh[:, 2 * H:])
            halves.append(n + z * (hp - n))
        if l < DEPTH - 1:
            h_scr[:, nl - 1:2 * nl - 1:2, :] = halves[0].reshape(B, m, H)
            h_scr[:, nl:2 * nl - 1:2, :] = halves[1].reshape(B, m, H)
        else:
            mL = jnp.max(halves[0].reshape(B, m, H), axis=1)
            mR = jnp.max(halves[1].reshape(B, m, H), axis=1)
            pooled = jnp.maximum(mL, mR)

    out_ref[0] = pooled


@functools.partial(jax.jit, static_argnames=())
def kernel(inputs, W_ih, W_hh, b_ih, b_hh, parent):
    del parent
    wih_t = W_ih.T
    whh_t = W_hh.T
    bih = b_ih.reshape(1, 3 * H)
    bhh = b_hh.reshape(1, 3 * H)

    grid = (T // B,)
    return pl.pallas_call(
        _tree_gru_kernel,
        grid=grid,
        in_specs=[
            pl.BlockSpec(memory_space=pl.ANY),
            pl.BlockSpec((IN, 3 * H), lambda i: (0, 0)),
            pl.BlockSpec((H, 3 * H), lambda i: (0, 0)),
            pl.BlockSpec((1, 3 * H), lambda i: (0, 0)),
            pl.BlockSpec((1, 3 * H), lambda i: (0, 0)),
        ],
        out_specs=pl.BlockSpec((1, B, H), lambda i: (i, 0, 0)),
        out_shape=jax.ShapeDtypeStruct((T // B, B, H), jnp.float32),
        scratch_shapes=[
            pltpu.VMEM((2, B, NPT, IN), jnp.float32),
            pltpu.VMEM((B, NPT, H), jnp.float32),
            pltpu.SemaphoreType.DMA((2,)),
        ],
        compiler_params=pltpu.CompilerParams(
            dimension_semantics=("arbitrary",),
        ),
    )(inputs, wih_t, whh_t, bih, bhh).reshape(T, H)
